# baseline (speedup 1.0000x reference)
import functools

import jax
import jax.numpy as jnp
from jax import lax
from jax.experimental import pallas as pl
from jax.experimental.pallas import tpu as pltpu
from jax.experimental.pallas import tpu_sc as plsc

B = 64
N = 32768
TWO_N = 2 * N
M = B * N
NC = 2
NS = 16
NW = NC * NS
BPW = B // NW
C = 2048
NCH = N // C
U = 4
GROUPS = C // 16


def _sl1x2_pair(d0, d1):
    ad0 = jnp.abs(d0)
    ad1 = jnp.abs(d1)
    m0 = jnp.minimum(ad0, 1.0)
    m1 = jnp.minimum(ad1, 1.0)
    s = ad0 + ad1
    q = m0 * (2.0 - m0) + m1 * (2.0 - m1)
    return (s + s) - q


_mesh = plsc.VectorSubcoreMesh(core_axis_name="c", subcore_axis_name="s")


@functools.partial(
    pl.kernel,
    out_type=jax.ShapeDtypeStruct((NW, 3, 16), jnp.float32),
    mesh=_mesh,
    compiler_params=pltpu.CompilerParams(needs_layout_passes=False),
    scratch_types=[
        pltpu.VMEM((TWO_N,), jnp.float32),
        pltpu.VMEM((2, 4, C), jnp.float32),
        pltpu.VMEM((3, 16), jnp.float32),
        pltpu.SemaphoreType.DMA,
        pltpu.SemaphoreType.DMA,
    ],
)
def _partials(x_hbm, t_hbm, out_hbm, xrow, tbuf, res, sem0, sem1):
    cid = lax.axis_index("c")
    sid = lax.axis_index("s")
    wid = sid * NC + cid
    iota = lax.broadcasted_iota(jnp.int32, (16,), 0)
    zeros = jnp.zeros((16,), jnp.float32)
    ones = jnp.ones((16,), jnp.float32)
    sems = (sem0, sem1)

    def row_group(s, base):
        t0 = tbuf[s, 0, pl.ds(base, 16)]
        t1 = tbuf[s, 1, pl.ds(base, 16)]
        ti = tbuf[s, 2, pl.ds(base, 16)]
        st = tbuf[s, 3, pl.ds(base, 16)]
        idx = ti.astype(jnp.int32)
        xlo = plsc.load_gather(xrow, [idx])
        xhi = plsc.load_gather(xrow, [idx + N])
        return _sl1x2_pair(xlo - t0, xhi - t1), st == 1.0

    def fire(b, c, s):
        return [
            pltpu.async_copy(t_hbm.at[j, b, pl.ds(c * C, C)], tbuf.at[s, j], sems[s])
            for j in range(4)
        ]

    acc = zeros
    cnt = zeros
    res[2] = zeros
    for i in range(BPW):
        b = wid * BPW + i
        pltpu.sync_copy(x_hbm.at[b], xrow)
        pending = fire(b, 0, 0)
        for c in range(NCH):
            s = c % 2
            nxt = fire(b, c + 1, 1 - s) if c + 1 < NCH else []
            for h in pending:
                h.wait()
            pending = nxt

            if i == 0 and c == 0:
                @pl.when(wid == 0)
                def _():
                    pair, _ = row_group(0, 0)
                    res[2] = jnp.where(iota == 0, pair, zeros)

            def group_body(g, carry, s=s):
                acc, cnt = carry
                for u in range(U):
                    pair, m = row_group(s, g * (16 * U) + u * 16)
                    w = jnp.where(m, ones, zeros)
                    acc = acc + pair * w
                    cnt = cnt + w
                return acc, cnt

            acc, cnt = lax.fori_loop(0, GROUPS // U, group_body, (acc, cnt))

    res[0] = acc
    res[1] = cnt
    pltpu.sync_copy(res, out_hbm.at[wid])


def kernel(input, target):
    t4 = jnp.transpose(jnp.reshape(target, (B, N, 4)), (2, 0, 1))
    parts = _partials(input, t4)
    s = jnp.sum(parts[:, 0, :])
    c = jnp.sum(parts[:, 1, :])
    p00 = parts[0, 2, 0]
    return 0.5 * (s + (jnp.float32(M) - c) * p00) / jnp.float32(M)


# speedup vs baseline: 1.0532x; 1.0532x over previous
import functools

import jax
import jax.numpy as jnp
from jax import lax
from jax.experimental import pallas as pl
from jax.experimental.pallas import tpu as pltpu
from jax.experimental.pallas import tpu_sc as plsc

B = 64
N = 32768
TWO_N = 2 * N
M = B * N
NC = 2
NS = 16
NW = NC * NS
BPW = B // NW
C = 4096
NCH = N // C
U = 4
GROUPS = C // 16


def _sl1x2_pair(d0, d1):
    ad0 = jnp.abs(d0)
    ad1 = jnp.abs(d1)
    m0 = jnp.minimum(ad0, 1.0)
    m1 = jnp.minimum(ad1, 1.0)
    s = ad0 + ad1
    q = m0 * (2.0 - m0) + m1 * (2.0 - m1)
    return (s + s) - q


_mesh = plsc.VectorSubcoreMesh(core_axis_name="c", subcore_axis_name="s")


@functools.partial(
    pl.kernel,
    out_type=jax.ShapeDtypeStruct((NW, 3, 16), jnp.float32),
    mesh=_mesh,
    compiler_params=pltpu.CompilerParams(needs_layout_passes=False),
    scratch_types=[
        pltpu.VMEM((TWO_N,), jnp.float32),
        pltpu.VMEM((2, 4, C), jnp.float32),
        pltpu.VMEM((3, 16), jnp.float32),
        pltpu.SemaphoreType.DMA,
        pltpu.SemaphoreType.DMA,
    ],
)
def _partials(x_hbm, t_hbm, out_hbm, xrow, tbuf, res, sem0, sem1):
    cid = lax.axis_index("c")
    sid = lax.axis_index("s")
    wid = sid * NC + cid
    iota = lax.broadcasted_iota(jnp.int32, (16,), 0)
    zeros = jnp.zeros((16,), jnp.float32)
    ones = jnp.ones((16,), jnp.float32)
    sems = (sem0, sem1)

    def row_group(s, base):
        t0 = tbuf[s, 0, pl.ds(base, 16)]
        t1 = tbuf[s, 1, pl.ds(base, 16)]
        ti = tbuf[s, 2, pl.ds(base, 16)]
        st = tbuf[s, 3, pl.ds(base, 16)]
        idx = ti.astype(jnp.int32)
        xlo = plsc.load_gather(xrow, [idx])
        xhi = plsc.load_gather(xrow, [idx + N])
        return _sl1x2_pair(xlo - t0, xhi - t1), st == 1.0

    def fire(b, c, s):
        return [
            pltpu.async_copy(t_hbm.at[j, b, pl.ds(c * C, C)], tbuf.at[s, j], sems[s])
            for j in range(4)
        ]

    acc = zeros
    cnt = zeros
    res[2] = zeros
    for i in range(BPW):
        b = wid * BPW + i
        pltpu.sync_copy(x_hbm.at[b], xrow)
        pending = fire(b, 0, 0)
        for c in range(NCH):
            s = c % 2
            nxt = fire(b, c + 1, 1 - s) if c + 1 < NCH else []
            for h in pending:
                h.wait()
            pending = nxt

            if i == 0 and c == 0:
                @pl.when(wid == 0)
                def _():
                    pair, _ = row_group(0, 0)
                    res[2] = jnp.where(iota == 0, pair, zeros)

            def group_body(g, carry, s=s):
                acc, cnt = carry
                for u in range(U):
                    pair, m = row_group(s, g * (16 * U) + u * 16)
                    w = jnp.where(m, ones, zeros)
                    acc = acc + pair * w
                    cnt = cnt + w
                return acc, cnt

            acc, cnt = lax.fori_loop(0, GROUPS // U, group_body, (acc, cnt))

    res[0] = acc
    res[1] = cnt
    pltpu.sync_copy(res, out_hbm.at[wid])


def kernel(input, target):
    t4 = jnp.transpose(jnp.reshape(target, (B, N, 4)), (2, 0, 1))
    parts = _partials(input, t4)
    s = jnp.sum(parts[:, 0, :])
    c = jnp.sum(parts[:, 1, :])
    p00 = parts[0, 2, 0]
    return 0.5 * (s + (jnp.float32(M) - c) * p00) / jnp.float32(M)
